# Initial kernel scaffold; baseline (speedup 1.0000x reference)
import jax
import jax.numpy as jnp
from jax.experimental import pallas as pl


def kernel(node_emb, edge_index, edge_vals, W1, b1, W2, b2):
    raise NotImplementedError("write your pallas kernel here")


# speedup vs baseline: 2.0537x; 2.0537x over previous
import functools

import jax
import jax.numpy as jnp
from jax import lax
from jax.experimental import pallas as pl
from jax.experimental.pallas import tpu as pltpu
from jax.experimental.pallas import tpu_sc as plsc

N = 10000
D = 128
H = 64
NW = 32
CH = 128
K = 40
EPAD = NW * K * CH
NLANE = 16


def _pq_body(ne_ref, w1_ref, b1_ref, p_ref, q_ref):
    x = ne_ref[...]
    w1 = w1_ref[...]
    p_ref[...] = lax.dot_general(x, w1[:D, :], (((1,), (0,)), ((), ())),
                                 preferred_element_type=jnp.float32)
    q_ref[...] = lax.dot_general(x, w1[D:, :], (((1,), (0,)), ((), ())),
                                 preferred_element_type=jnp.float32) + b1_ref[...]


def _compute_pq(node_emb, W1, b1):
    blk = 1000
    return pl.pallas_call(
        _pq_body,
        grid=(N // blk,),
        in_specs=[
            pl.BlockSpec((blk, D), lambda i: (i, 0)),
            pl.BlockSpec((2 * D, H), lambda i: (0, 0)),
            pl.BlockSpec((1, H), lambda i: (0, 0)),
        ],
        out_specs=[
            pl.BlockSpec((blk, H), lambda i: (i, 0)),
            pl.BlockSpec((blk, H), lambda i: (i, 0)),
        ],
        out_shape=[
            jax.ShapeDtypeStruct((N, H), jnp.float32),
            jax.ShapeDtypeStruct((N, H), jnp.float32),
        ],
    )(node_emb, W1, b1.reshape(1, H))


def _sc_edge_body(p_hbm, q_hbm, src_hbm, dst_hbm, nz_hbm, ev_hbm, w2_hbm,
                  out_hbm, psum_hbm,
                  srcv, dstv, nzv, evv, outv, w2v, psv,
                  pg0, pg1, qg0, qg1, sp0, sp1, sq0, sq1):
    wid = lax.axis_index("s") * 2 + lax.axis_index("c")
    pltpu.sync_copy(src_hbm.at[wid], srcv)
    pltpu.sync_copy(dst_hbm.at[wid], dstv)
    pltpu.sync_copy(nz_hbm.at[wid], nzv)
    pltpu.sync_copy(ev_hbm.at[wid], evv)
    pltpu.sync_copy(w2_hbm, w2v)
    w2rows = [w2v[pl.ds(j * NLANE, NLANE)] for j in range(H // NLANE)]

    def issue(t, pg, qg, sp, sq):
        pltpu.make_async_copy(p_hbm.at[srcv.at[t]], pg, sp).start()
        pltpu.make_async_copy(q_hbm.at[dstv.at[t]], qg, sq).start()

    def wait(t, pg, qg, sp, sq):
        pltpu.make_async_copy(p_hbm.at[srcv.at[t]], pg, sp).wait()
        pltpu.make_async_copy(q_hbm.at[dstv.at[t]], qg, sq).wait()

    def compute(t, pg, qg, psum):
        def gbody(g, psum):
            rows = g * NLANE + lax.iota(jnp.int32, NLANE)
            acc = jnp.zeros((NLANE,), jnp.float32)
            for f in range(H):
                fidx = jnp.full((NLANE,), f, jnp.int32)
                pv = plsc.load_gather(pg, [rows, fidx])
                qv = plsc.load_gather(qg, [rows, fidx])
                acc = acc + jnp.maximum(pv + qv, 0.0) * w2rows[f // NLANE][f % NLANE]
            nzg = nzv[t, pl.ds(g * NLANE, NLANE)]
            evg = evv[t, pl.ds(g * NLANE, NLANE)]
            aug = 1.0 / (1.0 + jnp.exp(-(acc + nzg)))
            outv[t, pl.ds(g * NLANE, NLANE)] = evg * aug
            return psum + aug
        return lax.fori_loop(0, CH // NLANE, gbody, psum)

    issue(0, pg0, qg0, sp0, sq0)

    def pair(i, psum):
        cc = 2 * i
        issue(cc + 1, pg1, qg1, sp1, sq1)
        wait(cc, pg0, qg0, sp0, sq0)
        psum = compute(cc, pg0, qg0, psum)

        @pl.when(i < K // 2 - 1)
        def _():
            issue(cc + 2, pg0, qg0, sp0, sq0)

        wait(cc + 1, pg1, qg1, sp1, sq1)
        psum = compute(cc + 1, pg1, qg1, psum)
        return psum

    psum = lax.fori_loop(0, K // 2, pair, jnp.zeros((NLANE,), jnp.float32))
    psv[...] = psum
    pltpu.sync_copy(outv, out_hbm.at[wid])
    pltpu.sync_copy(psv, psum_hbm.at[wid])


def _make_sc_call():
    mesh = plsc.VectorSubcoreMesh(core_axis_name="c", subcore_axis_name="s")
    return pl.kernel(
        _sc_edge_body,
        mesh=mesh,
        compiler_params=pltpu.CompilerParams(
            needs_layout_passes=False,
            use_tc_tiling_on_sc=False,
        ),
        out_type=[
            jax.ShapeDtypeStruct((NW, K, CH), jnp.float32),
            jax.ShapeDtypeStruct((NW, NLANE), jnp.float32),
        ],
        scratch_types=[
            pltpu.VMEM((K, CH), jnp.int32),
            pltpu.VMEM((K, CH), jnp.int32),
            pltpu.VMEM((K, CH), jnp.float32),
            pltpu.VMEM((K, CH), jnp.float32),
            pltpu.VMEM((K, CH), jnp.float32),
            pltpu.VMEM((H,), jnp.float32),
            pltpu.VMEM((NLANE,), jnp.float32),
            pltpu.VMEM((CH, H), jnp.float32),
            pltpu.VMEM((CH, H), jnp.float32),
            pltpu.VMEM((CH, H), jnp.float32),
            pltpu.VMEM((CH, H), jnp.float32),
            pltpu.SemaphoreType.DMA,
            pltpu.SemaphoreType.DMA,
            pltpu.SemaphoreType.DMA,
            pltpu.SemaphoreType.DMA,
        ],
    )


def kernel(node_emb, edge_index, edge_vals, W1, b1, W2, b2):
    half = edge_index.shape[1] // 2
    src = edge_index[0, :half]
    dst = edge_index[1, :half]

    p, q = _compute_pq(node_emb, W1, b1)

    bias = 0.0 + 0.0001
    u = jax.random.uniform(jax.random.key(42), (half, 1), dtype=jnp.float32)
    eps = (bias - (1.0 - bias)) * u + (1.0 - bias)
    noise = (jnp.log(eps) - jnp.log(1.0 - eps)).squeeze(-1)
    nz = noise + b2[0]

    pad = EPAD - half
    srcp = jnp.concatenate([src, jnp.zeros((pad,), jnp.int32)]).reshape(NW, K, CH)
    dstp = jnp.concatenate([dst, jnp.zeros((pad,), jnp.int32)]).reshape(NW, K, CH)
    nzp = jnp.concatenate([nz, jnp.full((pad,), -1e30, jnp.float32)]).reshape(NW, K, CH)
    evp = jnp.concatenate([edge_vals[:half], jnp.zeros((pad,), jnp.float32)]).reshape(NW, K, CH)

    outp, psum = _make_sc_call()(p, q, srcp, dstp, nzp, evp, W2.reshape(H))

    new_vals = outp.reshape(-1)[:half]
    sym_inds = jnp.concatenate([jnp.stack([src, dst]), jnp.stack([dst, src])], axis=1)
    sym_vals = jnp.concatenate([new_vals, new_vals], axis=0)
    mean_edge_weight = jnp.sum(psum) / half
    return (sym_inds, sym_vals, mean_edge_weight)


# speedup vs baseline: 2.3850x; 1.1613x over previous
import functools

import jax
import jax.numpy as jnp
from jax import lax
from jax.experimental import pallas as pl
from jax.experimental.pallas import tpu as pltpu
from jax.experimental.pallas import tpu_sc as plsc

N = 10000
D = 128
H = 64
NW = 32
CH = 128
K = 40
EPAD = NW * K * CH
NLANE = 16


def _pq_body(ne_ref, w1_ref, b1_ref, p_ref, q_ref):
    x = ne_ref[...]
    w1 = w1_ref[...]
    p_ref[...] = lax.dot_general(x, w1[:D, :], (((1,), (0,)), ((), ())),
                                 preferred_element_type=jnp.float32)
    q_ref[...] = lax.dot_general(x, w1[D:, :], (((1,), (0,)), ((), ())),
                                 preferred_element_type=jnp.float32) + b1_ref[...]


def _compute_pq(node_emb, W1, b1):
    blk = 1000
    return pl.pallas_call(
        _pq_body,
        grid=(N // blk,),
        in_specs=[
            pl.BlockSpec((blk, D), lambda i: (i, 0)),
            pl.BlockSpec((2 * D, H), lambda i: (0, 0)),
            pl.BlockSpec((1, H), lambda i: (0, 0)),
        ],
        out_specs=[
            pl.BlockSpec((blk, H), lambda i: (i, 0)),
            pl.BlockSpec((blk, H), lambda i: (i, 0)),
        ],
        out_shape=[
            jax.ShapeDtypeStruct((N, H), jnp.float32),
            jax.ShapeDtypeStruct((N, H), jnp.float32),
        ],
    )(node_emb, W1, b1.reshape(1, H))


def _sc_edge_body(p_hbm, q_hbm, src_hbm, dst_hbm, nz_hbm, ev_hbm, w2_hbm,
                  out_hbm, psum_hbm,
                  srcv, dstv, nzv, evv, outv, w2v, psv,
                  pg0, pg1, qg0, qg1, sp0, sp1, sq0, sq1):
    wid = lax.axis_index("s") * 2 + lax.axis_index("c")
    pltpu.sync_copy(src_hbm.at[wid], srcv)
    pltpu.sync_copy(dst_hbm.at[wid], dstv)
    pltpu.sync_copy(nz_hbm.at[wid], nzv)
    pltpu.sync_copy(ev_hbm.at[wid], evv)
    pltpu.sync_copy(w2_hbm, w2v)
    w2rows = [w2v[pl.ds(j * NLANE, NLANE)] for j in range(H // NLANE)]

    def issue(t, pg, qg, sp, sq):
        pltpu.make_async_copy(p_hbm.at[srcv.at[t]], pg, sp).start()
        pltpu.make_async_copy(q_hbm.at[dstv.at[t]], qg, sq).start()

    def wait(t, pg, qg, sp, sq):
        pltpu.make_async_copy(p_hbm.at[srcv.at[t]], pg, sp).wait()
        pltpu.make_async_copy(q_hbm.at[dstv.at[t]], qg, sq).wait()

    def compute(t, pg, qg, psum):
        def gbody(gg, psum):
            for half_g in range(2):
                g = gg * 2 + half_g
                rows = g * NLANE + lax.iota(jnp.int32, NLANE)
                accs = [jnp.zeros((NLANE,), jnp.float32) for _ in range(4)]
                for f in range(H):
                    fidx = jnp.full((NLANE,), f, jnp.int32)
                    pv = plsc.load_gather(pg, [rows, fidx])
                    qv = plsc.load_gather(qg, [rows, fidx])
                    w2f = w2rows[f // NLANE][f % NLANE]
                    accs[f % 4] = accs[f % 4] + jnp.maximum(pv + qv, 0.0) * w2f
                acc = (accs[0] + accs[1]) + (accs[2] + accs[3])
                nzg = nzv[t, pl.ds(g * NLANE, NLANE)]
                evg = evv[t, pl.ds(g * NLANE, NLANE)]
                aug = 1.0 / (1.0 + jnp.exp(-(acc + nzg)))
                outv[t, pl.ds(g * NLANE, NLANE)] = evg * aug
                psum = psum + aug
            return psum
        return lax.fori_loop(0, CH // NLANE // 2, gbody, psum)

    issue(0, pg0, qg0, sp0, sq0)

    def pair(i, psum):
        cc = 2 * i
        issue(cc + 1, pg1, qg1, sp1, sq1)
        wait(cc, pg0, qg0, sp0, sq0)
        psum = compute(cc, pg0, qg0, psum)

        @pl.when(i < K // 2 - 1)
        def _():
            issue(cc + 2, pg0, qg0, sp0, sq0)

        wait(cc + 1, pg1, qg1, sp1, sq1)
        psum = compute(cc + 1, pg1, qg1, psum)
        return psum

    psum = lax.fori_loop(0, K // 2, pair, jnp.zeros((NLANE,), jnp.float32))
    psv[...] = psum
    pltpu.sync_copy(outv, out_hbm.at[wid])
    pltpu.sync_copy(psv, psum_hbm.at[wid])


def _make_sc_call():
    mesh = plsc.VectorSubcoreMesh(core_axis_name="c", subcore_axis_name="s")
    return pl.kernel(
        _sc_edge_body,
        mesh=mesh,
        compiler_params=pltpu.CompilerParams(
            needs_layout_passes=False,
            use_tc_tiling_on_sc=False,
        ),
        out_type=[
            jax.ShapeDtypeStruct((NW, K, CH), jnp.float32),
            jax.ShapeDtypeStruct((NW, NLANE), jnp.float32),
        ],
        scratch_types=[
            pltpu.VMEM((K, CH), jnp.int32),
            pltpu.VMEM((K, CH), jnp.int32),
            pltpu.VMEM((K, CH), jnp.float32),
            pltpu.VMEM((K, CH), jnp.float32),
            pltpu.VMEM((K, CH), jnp.float32),
            pltpu.VMEM((H,), jnp.float32),
            pltpu.VMEM((NLANE,), jnp.float32),
            pltpu.VMEM((CH, H), jnp.float32),
            pltpu.VMEM((CH, H), jnp.float32),
            pltpu.VMEM((CH, H), jnp.float32),
            pltpu.VMEM((CH, H), jnp.float32),
            pltpu.SemaphoreType.DMA,
            pltpu.SemaphoreType.DMA,
            pltpu.SemaphoreType.DMA,
            pltpu.SemaphoreType.DMA,
        ],
    )


def kernel(node_emb, edge_index, edge_vals, W1, b1, W2, b2):
    half = edge_index.shape[1] // 2
    src = edge_index[0, :half]
    dst = edge_index[1, :half]

    p, q = _compute_pq(node_emb, W1, b1)

    bias = 0.0 + 0.0001
    u = jax.random.uniform(jax.random.key(42), (half, 1), dtype=jnp.float32)
    eps = (bias - (1.0 - bias)) * u + (1.0 - bias)
    noise = (jnp.log(eps) - jnp.log(1.0 - eps)).squeeze(-1)
    nz = noise + b2[0]

    pad = EPAD - half
    srcp = jnp.concatenate([src, jnp.zeros((pad,), jnp.int32)]).reshape(NW, K, CH)
    dstp = jnp.concatenate([dst, jnp.zeros((pad,), jnp.int32)]).reshape(NW, K, CH)
    nzp = jnp.concatenate([nz, jnp.full((pad,), -1e30, jnp.float32)]).reshape(NW, K, CH)
    evp = jnp.concatenate([edge_vals[:half], jnp.zeros((pad,), jnp.float32)]).reshape(NW, K, CH)

    outp, psum = _make_sc_call()(p, q, srcp, dstp, nzp, evp, W2.reshape(H))

    new_vals = outp.reshape(-1)[:half]
    sym_inds = jnp.concatenate([jnp.stack([src, dst]), jnp.stack([dst, src])], axis=1)
    sym_vals = jnp.concatenate([new_vals, new_vals], axis=0)
    mean_edge_weight = jnp.sum(psum) / half
    return (sym_inds, sym_vals, mean_edge_weight)


# speedup vs baseline: 3.9899x; 1.6730x over previous
import functools

import jax
import jax.numpy as jnp
from jax import lax
from jax.experimental import pallas as pl
from jax.experimental.pallas import tpu as pltpu
from jax.experimental.pallas import tpu_sc as plsc

N = 10000
D = 128
H = 64
NW = 32
CH = 128
K = 40
EPAD = NW * K * CH
NLANE = 16
HP = H + 1


def _pq_body(ne_ref, w1_ref, b1_ref, p_ref, q_ref):
    x = ne_ref[...]
    w1 = w1_ref[...]
    p_ref[...] = lax.dot_general(x, w1[:D, :], (((1,), (0,)), ((), ())),
                                 preferred_element_type=jnp.float32)
    q_ref[...] = lax.dot_general(x, w1[D:, :], (((1,), (0,)), ((), ())),
                                 preferred_element_type=jnp.float32) + b1_ref[...]


def _compute_pq(node_emb, W1, b1):
    blk = 1000
    return pl.pallas_call(
        _pq_body,
        grid=(N // blk,),
        in_specs=[
            pl.BlockSpec((blk, D), lambda i: (i, 0)),
            pl.BlockSpec((2 * D, H), lambda i: (0, 0)),
            pl.BlockSpec((1, H), lambda i: (0, 0)),
        ],
        out_specs=[
            pl.BlockSpec((blk, H), lambda i: (i, 0)),
            pl.BlockSpec((blk, H), lambda i: (i, 0)),
        ],
        out_shape=[
            jax.ShapeDtypeStruct((N, H), jnp.float32),
            jax.ShapeDtypeStruct((N, H), jnp.float32),
        ],
    )(node_emb, W1, b1.reshape(1, H))


NBUF = 4


def _sc_edge_body(p_hbm, q_hbm, src_hbm, dst_hbm, nz_hbm, ev_hbm, w2_hbm,
                  out_hbm, psum_hbm,
                  srcv, dstv, nzv, evv, outv, w2v, psv,
                  *bufs):
    pgs = bufs[0:NBUF]
    qgs = bufs[NBUF:2 * NBUF]
    sps = bufs[2 * NBUF:3 * NBUF]
    sqs = bufs[3 * NBUF:4 * NBUF]
    wid = lax.axis_index("s") * 2 + lax.axis_index("c")
    pltpu.sync_copy(src_hbm.at[wid], srcv)
    pltpu.sync_copy(dst_hbm.at[wid], dstv)
    pltpu.sync_copy(nz_hbm.at[wid], nzv)
    pltpu.sync_copy(ev_hbm.at[wid], evv)
    pltpu.sync_copy(w2_hbm, w2v)
    w2rows = [w2v[pl.ds(j * NLANE, NLANE)] for j in range(H // NLANE)]

    def issue(t, pg, qg, sp, sq):
        pltpu.make_async_copy(p_hbm.at[srcv.at[t]], pg, sp).start()
        pltpu.make_async_copy(q_hbm.at[dstv.at[t]], qg, sq).start()

    def wait(t, pg, qg, sp, sq):
        pltpu.make_async_copy(p_hbm.at[srcv.at[t]], pg, sp).wait()
        pltpu.make_async_copy(q_hbm.at[dstv.at[t]], qg, sq).wait()

    def compute(t, pg, qg, psum):
        def gbody(gg, psum):
            for half_g in range(2):
                g = gg * 2 + half_g
                rows = g * NLANE + lax.iota(jnp.int32, NLANE)
                accs = [jnp.zeros((NLANE,), jnp.float32) for _ in range(4)]
                for f in range(H):
                    fidx = jnp.full((NLANE,), f, jnp.int32)
                    pv = plsc.load_gather(pg, [rows, fidx])
                    qv = plsc.load_gather(qg, [rows, fidx])
                    w2f = w2rows[f // NLANE][f % NLANE]
                    accs[f % 4] = accs[f % 4] + jnp.maximum(pv + qv, 0.0) * w2f
                acc = (accs[0] + accs[1]) + (accs[2] + accs[3])
                nzg = nzv[t, pl.ds(g * NLANE, NLANE)]
                evg = evv[t, pl.ds(g * NLANE, NLANE)]
                aug = 1.0 / (1.0 + jnp.exp(-(acc + nzg)))
                outv[t, pl.ds(g * NLANE, NLANE)] = evg * aug
                psum = psum + aug
            return psum
        return lax.fori_loop(0, CH // NLANE // 2, gbody, psum)

    for b in range(NBUF - 1):
        issue(b, pgs[b], qgs[b], sps[b], sqs[b])

    def quad(i, psum):
        t0 = NBUF * i
        for b in range(NBUF):
            t = t0 + b
            wait(t, pgs[b], qgs[b], sps[b], sqs[b])
            psum = compute(t, pgs[b], qgs[b], psum)
            b2 = (b + NBUF - 1) % NBUF

            @pl.when(t + NBUF - 1 < K)
            def _():
                issue(t + NBUF - 1, pgs[b2], qgs[b2], sps[b2], sqs[b2])

        return psum

    psum = lax.fori_loop(0, K // NBUF, quad, jnp.zeros((NLANE,), jnp.float32))
    psv[...] = psum
    pltpu.sync_copy(outv, out_hbm.at[wid])
    pltpu.sync_copy(psv, psum_hbm.at[wid])


def _make_sc_call():
    mesh = plsc.VectorSubcoreMesh(core_axis_name="c", subcore_axis_name="s")
    return pl.kernel(
        _sc_edge_body,
        mesh=mesh,
        compiler_params=pltpu.CompilerParams(
            needs_layout_passes=False,
            use_tc_tiling_on_sc=False,
        ),
        out_type=[
            jax.ShapeDtypeStruct((NW, K, CH), jnp.float32),
            jax.ShapeDtypeStruct((NW, NLANE), jnp.float32),
        ],
        scratch_types=[
            pltpu.VMEM((K, CH), jnp.int32),
            pltpu.VMEM((K, CH), jnp.int32),
            pltpu.VMEM((K, CH), jnp.float32),
            pltpu.VMEM((K, CH), jnp.float32),
            pltpu.VMEM((K, CH), jnp.float32),
            pltpu.VMEM((H,), jnp.float32),
            pltpu.VMEM((NLANE,), jnp.float32),
            *[pltpu.VMEM((CH, HP), jnp.float32) for _ in range(2 * NBUF)],
            *[pltpu.SemaphoreType.DMA for _ in range(2 * NBUF)],
        ],
    )


def kernel(node_emb, edge_index, edge_vals, W1, b1, W2, b2):
    half = edge_index.shape[1] // 2
    src = edge_index[0, :half]
    dst = edge_index[1, :half]

    p, q = _compute_pq(node_emb, W1, b1)
    p = jnp.pad(p, ((0, 0), (0, HP - H)))
    q = jnp.pad(q, ((0, 0), (0, HP - H)))

    bias = 0.0 + 0.0001
    u = jax.random.uniform(jax.random.key(42), (half, 1), dtype=jnp.float32)
    eps = (bias - (1.0 - bias)) * u + (1.0 - bias)
    noise = (jnp.log(eps) - jnp.log(1.0 - eps)).squeeze(-1)
    nz = noise + b2[0]

    pad = EPAD - half
    srcp = jnp.concatenate([src, jnp.zeros((pad,), jnp.int32)]).reshape(NW, K, CH)
    dstp = jnp.concatenate([dst, jnp.zeros((pad,), jnp.int32)]).reshape(NW, K, CH)
    nzp = jnp.concatenate([nz, jnp.full((pad,), -1e30, jnp.float32)]).reshape(NW, K, CH)
    evp = jnp.concatenate([edge_vals[:half], jnp.zeros((pad,), jnp.float32)]).reshape(NW, K, CH)

    outp, psum = _make_sc_call()(p, q, srcp, dstp, nzp, evp, W2.reshape(H))

    new_vals = outp.reshape(-1)[:half]
    sym_inds = jnp.concatenate([jnp.stack([src, dst]), jnp.stack([dst, src])], axis=1)
    sym_vals = jnp.concatenate([new_vals, new_vals], axis=0)
    mean_edge_weight = jnp.sum(psum) / half
    return (sym_inds, sym_vals, mean_edge_weight)
